# baseline (speedup 1.0000x reference)
import functools

import jax
import jax.numpy as jnp
from jax import lax
from jax.experimental import pallas as pl
from jax.experimental.pallas import tpu as pltpu, tpu_sc as plsc

B, L, H = 16, 512, 16
NS = 513
SUPER = NS - 1
TBL = H * NS
NW = 32
ROWS = B * L
RPW = ROWS // NW
CROWS = 4
NCHUNK = RPW // CROWS


@functools.partial(
    pl.kernel,
    out_type=jax.ShapeDtypeStruct((B, H, L, L), jnp.float32),
    mesh=plsc.VectorSubcoreMesh(core_axis_name="c", subcore_axis_name="s"),
    compiler_params=pltpu.CompilerParams(needs_layout_passes=False),
    scratch_types=[
        pltpu.VMEM((TBL,), jnp.float32),
        pltpu.VMEM((2, CROWS, L), jnp.int32),
        pltpu.VMEM((2, H, CROWS, L), jnp.float32),
        pltpu.SemaphoreType.DMA,
        pltpu.SemaphoreType.DMA,
        pltpu.SemaphoreType.DMA,
        pltpu.SemaphoreType.DMA,
    ],
)
def _sc_lookup(sp_hbm, emb_hbm, out_hbm, tbl_v, idx_v, obuf_v,
               isem0, isem1, osem0, osem1):
    wid = lax.axis_index("s") * 2 + lax.axis_index("c")
    pltpu.sync_copy(emb_hbm, tbl_v)
    lane = lax.broadcasted_iota(jnp.int32, (16,), 0)
    row0 = wid * RPW
    b = row0 // L
    i0 = row0 % L
    isems = (isem0, isem1)
    osems = (osem0, osem1)

    pltpu.make_async_copy(
        sp_hbm.at[b, pl.ds(i0, CROWS), :], idx_v.at[0], isem0).start()

    def pair(p, carry):
        for par in (0, 1):
            c = 2 * p + par
            i_start = i0 + c * CROWS

            @pl.when(c + 1 < NCHUNK)
            def _():
                pltpu.make_async_copy(
                    sp_hbm.at[b, pl.ds(i_start + CROWS, CROWS), :],
                    idx_v.at[1 - par], isems[1 - par]).start()

            pltpu.make_async_copy(
                sp_hbm.at[b, pl.ds(0, CROWS), :],
                idx_v.at[par], isems[par]).wait()

            @pl.when(c >= 2)
            def _():
                pltpu.make_async_copy(
                    obuf_v.at[par],
                    out_hbm.at[0, :, pl.ds(0, CROWS), :], osems[par]).wait()

            for r in range(CROWS):
                v = idx_v[par, r, pl.ds(0, 16)]
                idx_v[par, r, pl.ds(0, 16)] = jnp.where(lane == 0, SUPER, v)

            @pl.when((i0 == 0) & (c == 0))
            def _():
                for g in range(L // 16):
                    idx_v[par, 0, pl.ds(g * 16, 16)] = jnp.full(
                        (16,), SUPER, jnp.int32)

            for r in range(CROWS):
                @plsc.parallel_loop(0, L // 16, unroll=8)
                def _(g):
                    col = g * 16
                    ivec = idx_v[par, r, pl.ds(col, 16)]
                    for h in range(H):
                        obuf_v[par, h, r, pl.ds(col, 16)] = plsc.load_gather(
                            tbl_v, [ivec + h * NS])

            for h in range(H):
                pltpu.make_async_copy(
                    obuf_v.at[par, h],
                    out_hbm.at[b, h, pl.ds(i_start, CROWS), :],
                    osems[par]).start()
        return carry

    lax.fori_loop(0, NCHUNK // 2, pair, 0)

    for par in (0, 1):
        pltpu.make_async_copy(
            obuf_v.at[par],
            out_hbm.at[0, :, pl.ds(0, CROWS), :], osems[par]).wait()


def kernel(spatial_pos, emb):
    emb_t = jnp.transpose(emb).reshape(-1)
    return _sc_lookup(spatial_pos, emb_t)


# speedup vs baseline: 1.1949x; 1.1949x over previous
import functools

import jax
import jax.numpy as jnp
from jax import lax
from jax.experimental import pallas as pl
from jax.experimental.pallas import tpu as pltpu, tpu_sc as plsc

B, L, H = 16, 512, 16
NS = 513
SUPER = NS - 1
TBL = H * NS
NW = 32
ROWS = B * L
RPW = ROWS // NW
CROWS = 4
NCHUNK = RPW // CROWS


@functools.partial(
    pl.kernel,
    out_type=jax.ShapeDtypeStruct((B, H, L, L), jnp.float32),
    mesh=plsc.VectorSubcoreMesh(core_axis_name="c", subcore_axis_name="s"),
    compiler_params=pltpu.CompilerParams(needs_layout_passes=False),
    scratch_types=[
        pltpu.VMEM((TBL,), jnp.float32),
        pltpu.VMEM((2, CROWS, L), jnp.int32),
        pltpu.VMEM((2, H, CROWS, L), jnp.float32),
        pltpu.SemaphoreType.DMA,
        pltpu.SemaphoreType.DMA,
        pltpu.SemaphoreType.DMA,
        pltpu.SemaphoreType.DMA,
    ],
)
def _sc_lookup(sp_hbm, emb_hbm, out_hbm, tbl_v, idx_v, obuf_v,
               isem0, isem1, osem0, osem1):
    wid = lax.axis_index("s") * 2 + lax.axis_index("c")
    pltpu.sync_copy(emb_hbm, tbl_v)
    lane = lax.broadcasted_iota(jnp.int32, (16,), 0)
    row0 = wid * RPW
    b = row0 // L
    i0 = row0 % L
    isems = (isem0, isem1)
    osems = (osem0, osem1)

    pltpu.make_async_copy(
        sp_hbm.at[b, pl.ds(i0, CROWS), :], idx_v.at[0], isem0).start()

    def pair(p, carry):
        for par in (0, 1):
            c = 2 * p + par
            i_start = i0 + c * CROWS

            @pl.when(c + 1 < NCHUNK)
            def _():
                pltpu.make_async_copy(
                    sp_hbm.at[b, pl.ds(i_start + CROWS, CROWS), :],
                    idx_v.at[1 - par], isems[1 - par]).start()

            pltpu.make_async_copy(
                sp_hbm.at[b, pl.ds(0, CROWS), :],
                idx_v.at[par], isems[par]).wait()

            @pl.when(c >= 2)
            def _():
                pltpu.make_async_copy(
                    obuf_v.at[par],
                    out_hbm.at[0, :, pl.ds(0, CROWS), :], osems[par]).wait()

            for r in range(CROWS):
                v = idx_v[par, r, pl.ds(0, 16)]
                idx_v[par, r, pl.ds(0, 16)] = jnp.where(lane == 0, SUPER, v)

            @pl.when((i0 == 0) & (c == 0))
            def _():
                for g in range(L // 16):
                    idx_v[par, 0, pl.ds(g * 16, 16)] = jnp.full(
                        (16,), SUPER, jnp.int32)

            for r in range(CROWS):
                @plsc.parallel_loop(0, L // 16, unroll=2)
                def _(g):
                    col = g * 16
                    ivec = idx_v[par, r, pl.ds(col, 16)]
                    for h in range(H):
                        obuf_v[par, h, r, pl.ds(col, 16)] = plsc.load_gather(
                            tbl_v, [ivec + h * NS])

            for h in range(H):
                pltpu.make_async_copy(
                    obuf_v.at[par, h],
                    out_hbm.at[b, h, pl.ds(i_start, CROWS), :],
                    osems[par]).start()
        return carry

    lax.fori_loop(0, NCHUNK // 2, pair, 0)

    for par in (0, 1):
        pltpu.make_async_copy(
            obuf_v.at[par],
            out_hbm.at[0, :, pl.ds(0, CROWS), :], osems[par]).wait()


def kernel(spatial_pos, emb):
    emb_t = jnp.transpose(emb).reshape(-1)
    return _sc_lookup(spatial_pos, emb_t)


# speedup vs baseline: 1.3021x; 1.0898x over previous
import functools

import jax
import jax.numpy as jnp
from jax import lax
from jax.experimental import pallas as pl
from jax.experimental.pallas import tpu as pltpu, tpu_sc as plsc

B, L, H = 16, 512, 16
NS = 513
SUPER = NS - 1
TBL = H * NS
NW = 32
ROWS = B * L
RPW = ROWS // NW
CROWS = 4
NCHUNK = RPW // CROWS


@functools.partial(
    pl.kernel,
    out_type=jax.ShapeDtypeStruct((B, H, L, L), jnp.float32),
    mesh=plsc.VectorSubcoreMesh(core_axis_name="c", subcore_axis_name="s"),
    compiler_params=pltpu.CompilerParams(needs_layout_passes=False),
    scratch_types=[
        pltpu.VMEM((TBL,), jnp.float32),
        pltpu.VMEM((2, CROWS, L), jnp.int32),
        pltpu.VMEM((2, H, CROWS, L), jnp.float32),
        pltpu.SemaphoreType.DMA,
        pltpu.SemaphoreType.DMA,
        pltpu.SemaphoreType.DMA,
        pltpu.SemaphoreType.DMA,
    ],
)
def _sc_lookup(sp_hbm, emb_hbm, out_hbm, tbl_v, idx_v, obuf_v,
               isem0, isem1, osem0, osem1):
    wid = lax.axis_index("s") * 2 + lax.axis_index("c")
    pltpu.sync_copy(emb_hbm, tbl_v)
    lane = lax.broadcasted_iota(jnp.int32, (16,), 0)
    row0 = wid * RPW
    b = row0 // L
    i0 = row0 % L
    isems = (isem0, isem1)
    osems = (osem0, osem1)

    pltpu.make_async_copy(
        sp_hbm.at[b, pl.ds(i0, CROWS), :], idx_v.at[0], isem0).start()

    def pair(p, carry):
        for par in (0, 1):
            c = 2 * p + par
            i_start = i0 + c * CROWS

            @pl.when(c + 1 < NCHUNK)
            def _():
                pltpu.make_async_copy(
                    sp_hbm.at[b, pl.ds(i_start + CROWS, CROWS), :],
                    idx_v.at[1 - par], isems[1 - par]).start()

            pltpu.make_async_copy(
                sp_hbm.at[b, pl.ds(0, CROWS), :],
                idx_v.at[par], isems[par]).wait()

            @pl.when(c >= 2)
            def _():
                pltpu.make_async_copy(
                    obuf_v.at[par],
                    out_hbm.at[0, :, pl.ds(0, CROWS), :], osems[par]).wait()

            for r in range(CROWS):
                v = idx_v[par, r, pl.ds(0, 16)]
                idx_v[par, r, pl.ds(0, 16)] = jnp.where(lane == 0, SUPER, v)

            @pl.when((i0 == 0) & (c == 0))
            def _():
                for g in range(L // 16):
                    idx_v[par, 0, pl.ds(g * 16, 16)] = jnp.full(
                        (16,), SUPER, jnp.int32)

            for r in range(CROWS):
                @plsc.parallel_loop(0, L // 16, unroll=1)
                def _(g):
                    col = g * 16
                    ivec = idx_v[par, r, pl.ds(col, 16)]
                    for h in range(H):
                        obuf_v[par, h, r, pl.ds(col, 16)] = plsc.load_gather(
                            tbl_v, [ivec + h * NS])

            for h in range(H):
                pltpu.make_async_copy(
                    obuf_v.at[par, h],
                    out_hbm.at[b, h, pl.ds(i_start, CROWS), :],
                    osems[par]).start()
        return carry

    lax.fori_loop(0, NCHUNK // 2, pair, 0)

    for par in (0, 1):
        pltpu.make_async_copy(
            obuf_v.at[par],
            out_hbm.at[0, :, pl.ds(0, CROWS), :], osems[par]).wait()


def kernel(spatial_pos, emb):
    emb_t = jnp.transpose(emb).reshape(-1)
    return _sc_lookup(spatial_pos, emb_t)
